# baseline (speedup 1.0000x reference)
import jax
import jax.numpy as jnp
from jax import lax
from jax.experimental import pallas as pl
from jax.experimental.pallas import tpu as pltpu
from jax.experimental.pallas import tpu_sc as plsc

N = 10000
E = 160000
D = 256
DE = 16
H = 128
L = 4
LAT = 128
G = 64

NC = 2
NS = 16
NW = NC * NS
CHUNK = 128
E_PAD = 163840
N_PAD = 10240
ZPW = N_PAD // (NS * CHUNK)

SCH = 64
CN = E_PAD // (NW * SCH)
ZPS = N_PAD // (NS * SCH)
SPC = 128 // SCH

SPW = E_PAD // (NW * 64)
ZPSD = N_PAD // (NS * 64)

NBUF = 4

NB = 1000
EB = 2048
F32 = jnp.float32


def _silu(v):
    return v * jax.nn.sigmoid(v)


def _sc_edge_body(a_hbm, b_hbm, ce_hbm, srcp, tgtu, agg_out,
                  idx_s, idx_t, bufA, shared,
                  sem_l, sem_ga, sem_gb, sem_s, sem_w):
    cid = lax.axis_index("c")
    sid = lax.axis_index("s")
    wid = sid * NC + cid
    base = wid * CN

    def zrow(r, _):
        for q in range(H // 16):
            bufA[0, r, pl.ds(q * 16, 16)] = jnp.zeros((16,), F32)
        return _

    lax.fori_loop(0, SCH, zrow, None)
    for k in range(ZPS):
        pltpu.sync_copy(bufA.at[0],
                        shared.at[pl.ds((sid * ZPS + k) * SCH, SCH)])
    pltpu.sync_copy(srcp.at[pl.ds(wid * (CN // SPC), CN // SPC)], idx_s)
    pltpu.sync_copy(tgtu.at[pl.ds(base, CN)], idx_t)
    plsc.subcore_barrier()

    def s_wait(b):
        pltpu.make_async_copy(bufA.at[b], shared.at[pl.ds(0, SCH)],
                              sem_s.at[b]).wait()

    def st0(b, c):
        off = (base + c) * SCH
        pltpu.async_copy(ce_hbm.at[pl.ds(off, SCH)], bufA.at[b], sem_l.at[b])

    def st1(b, c):
        pltpu.make_async_copy(ce_hbm.at[pl.ds(0, SCH)], bufA.at[b],
                              sem_l.at[b]).wait()
        pltpu.async_copy(
            a_hbm.at[idx_s.at[c // SPC, pl.ds((c % SPC) * SCH, SCH)]],
            bufA.at[b], sem_ga.at[b], add=True)

    def st2(b, c):
        pltpu.make_async_copy(ce_hbm.at[pl.ds(0, SCH)], bufA.at[b],
                              sem_ga.at[b]).wait()
        pltpu.async_copy(b_hbm.at[idx_t.at[c]], bufA.at[b], sem_gb.at[b],
                         add=True)

    def st3(b, c):
        pltpu.make_async_copy(ce_hbm.at[pl.ds(0, SCH)], bufA.at[b],
                              sem_gb.at[b]).wait()

        def row(r, _):
            for q in range(H // 16):
                sl = pl.ds(q * 16, 16)
                z = bufA[b, r, sl]
                bufA[b, r, sl] = z / (1.0 + jnp.exp(-z))
            return _

        lax.fori_loop(0, SCH, row, None)
        pltpu.async_copy(bufA.at[b], shared.at[idx_t.at[c]], sem_s.at[b],
                         add=True)

    def step(s, _):
        for b in range(NBUF):
            t = s * NBUF + b

            @pl.when(t >= NBUF)
            def _():
                s_wait(b)
            st0(b, t)

            @pl.when(t >= 1)
            def _():
                st1((b + NBUF - 1) % NBUF, t - 1)

            @pl.when(t >= 2)
            def _():
                st2((b + NBUF - 2) % NBUF, t - 2)

            @pl.when(t >= 3)
            def _():
                st3((b + NBUF - 3) % NBUF, t - 3)
        return _

    lax.fori_loop(0, CN // NBUF, step, None)
    st1((CN - 1) % NBUF, CN - 1)
    st2((CN - 2) % NBUF, CN - 2)
    st2((CN - 1) % NBUF, CN - 1)
    for u in (CN - 3, CN - 2, CN - 1):
        st3(u % NBUF, u)
    for b in range(NBUF):
        s_wait(b)

    plsc.subcore_barrier()
    for k in range(ZPW):
        row = (sid * ZPW + k) * CHUNK
        pltpu.async_copy(shared.at[pl.ds(row, CHUNK)],
                         agg_out.at[cid, pl.ds(row, CHUNK)], sem_w)
    for k in range(ZPW):
        pltpu.make_async_copy(shared.at[pl.ds(0, CHUNK)],
                              agg_out.at[cid, pl.ds(0, CHUNK)], sem_w).wait()


@jax.jit
def _edge_sc(a_tab, b_tab, ce, srcp, tgtu):
    return pl.kernel(
        _sc_edge_body,
        out_type=jax.ShapeDtypeStruct((NC, N_PAD, H), F32),
        mesh=plsc.VectorSubcoreMesh(core_axis_name="c", subcore_axis_name="s"),
        scratch_types=[
            pltpu.VMEM((CN // SPC, SPC * SCH), jnp.int32),
            pltpu.VMEM((CN, SCH), jnp.int32),
            pltpu.VMEM((NBUF, SCH, H), F32),
            pltpu.VMEM_SHARED((N_PAD, H), F32),
            pltpu.SemaphoreType.DMA((NBUF,)),
            pltpu.SemaphoreType.DMA((NBUF,)),
            pltpu.SemaphoreType.DMA((NBUF,)),
            pltpu.SemaphoreType.DMA((NBUF,)),
            pltpu.SemaphoreType.DMA,
        ],
    )(a_tab, b_tab, ce, srcp, tgtu)


def _sc_deg_body(tgt_hbm, agg_out, idx_t, buf, shared, sem_a, sem_w):
    cid = lax.axis_index("c")
    sid = lax.axis_index("s")
    wid = sid * NC + cid
    base = wid * SPW

    def fill(v, row_buf):
        def zrow(r, _):
            for q in range(H // 16):
                buf[row_buf, r, pl.ds(q * 16, 16)] = jnp.full((16,), v, F32)
            return _
        lax.fori_loop(0, 64, zrow, None)

    fill(0.0, 0)
    for k in range(ZPSD):
        pltpu.sync_copy(buf.at[0], shared.at[pl.ds((sid * ZPSD + k) * 64, 64)])
    fill(1.0, 1)
    pltpu.sync_copy(tgt_hbm.at[pl.ds(base, SPW)], idx_t)
    plsc.subcore_barrier()

    def a_start(b, c):
        pltpu.async_copy(buf.at[1], shared.at[idx_t.at[c]], sem_a.at[b],
                         add=True)

    def a_wait(b):
        pltpu.make_async_copy(buf.at[1], shared.at[pl.ds(0, 64)],
                              sem_a.at[b]).wait()

    for b in range(NBUF):
        a_start(b, b)

    def step(s, _):
        for b in range(NBUF):
            c = s * NBUF + b

            @pl.when(c >= NBUF)
            def _():
                a_wait(b)
                a_start(b, c)
        return _

    lax.fori_loop(1, SPW // NBUF, step, None)
    for b in range(NBUF):
        a_wait(b)

    plsc.subcore_barrier()
    for k in range(ZPW):
        row = (sid * ZPW + k) * CHUNK
        pltpu.async_copy(shared.at[pl.ds(row, CHUNK)],
                         agg_out.at[cid, pl.ds(row, CHUNK)], sem_w)
    for k in range(ZPW):
        pltpu.make_async_copy(shared.at[pl.ds(0, CHUNK)],
                              agg_out.at[cid, pl.ds(0, CHUNK)], sem_w).wait()


@jax.jit
def _scatter_ones(tgt2s):
    return pl.kernel(
        _sc_deg_body,
        out_type=jax.ShapeDtypeStruct((NC, N_PAD, H), F32),
        mesh=plsc.VectorSubcoreMesh(core_axis_name="c", subcore_axis_name="s"),
        scratch_types=[
            pltpu.VMEM((SPW, 64), jnp.int32),
            pltpu.VMEM((2, 64, H), F32),
            pltpu.VMEM_SHARED((N_PAD, H), F32),
            pltpu.SemaphoreType.DMA((NBUF,)),
            pltpu.SemaphoreType.DMA,
        ],
    )(tgt2s)


def _mlp2_body(x_ref, w1_ref, b1_ref, w2_ref, b2_ref, o_ref):
    a = jnp.dot(x_ref[...], w1_ref[...], preferred_element_type=F32) + b1_ref[...]
    o_ref[...] = (jnp.dot(_silu(a), w2_ref[...], preferred_element_type=F32)
                  + b2_ref[...])


def _mlp2(x, w1, b1, w2, b2, blk):
    n, d = x.shape
    h1 = w1.shape[1]
    h2 = w2.shape[1]
    return pl.pallas_call(
        _mlp2_body,
        grid=(n // blk,),
        in_specs=[
            pl.BlockSpec((blk, d), lambda i: (i, 0)),
            pl.BlockSpec((d, h1), lambda i: (0, 0)),
            pl.BlockSpec((1, h1), lambda i: (0, 0)),
            pl.BlockSpec((h1, h2), lambda i: (0, 0)),
            pl.BlockSpec((1, h2), lambda i: (0, 0)),
        ],
        out_specs=pl.BlockSpec((blk, h2), lambda i: (i, 0)),
        out_shape=jax.ShapeDtypeStruct((n, h2), F32),
    )(x, w1, b1.reshape(1, -1), w2, b2.reshape(1, -1))


def _wprep_body(mw1_ref, mb1_ref, mw2_ref, mb2_ref, uw1_ref, ew2_ref,
                eb2_ref, wt_ref, bt_ref, p_ref, q_ref):
    for i in range(L):
        w1c = mw1_ref[i, 2 * H:, :]
        u1b = uw1_ref[i, H:, :]
        wt_ref[i, :, :] = jnp.dot(ew2_ref[...], w1c,
                                  preferred_element_type=F32)
        bt_ref[i:i + 1, :] = (jnp.dot(eb2_ref[...], w1c,
                                      preferred_element_type=F32)
                              + mb1_ref[i:i + 1, :])
        p_ref[i, :, :] = jnp.dot(mw2_ref[i, :, :], u1b,
                                 preferred_element_type=F32)
        q_ref[i:i + 1, :] = jnp.dot(mb2_ref[i:i + 1, :], u1b,
                                    preferred_element_type=F32)


@jax.jit
def _weight_prep(msg_w1, msg_b1, msg_w2, msg_b2, upd_w1, edge_w2, edge_b2):
    return pl.pallas_call(
        _wprep_body,
        out_shape=(jax.ShapeDtypeStruct((L, H, H), F32),
                   jax.ShapeDtypeStruct((L, H), F32),
                   jax.ShapeDtypeStruct((L, H, H), F32),
                   jax.ShapeDtypeStruct((L, H), F32)),
    )(msg_w1, msg_b1, msg_w2, msg_b2, upd_w1, edge_w2,
      edge_b2.reshape(1, -1))


def _edge_cs_body(ea_ref, w1_ref, b1_ref, wt_ref, bt_ref, o0, o1, o2, o3):
    z2 = _silu(jnp.dot(ea_ref[...], w1_ref[...], preferred_element_type=F32)
               + b1_ref[...])
    gbase = pl.program_id(0) * EB
    pad = (gbase + lax.broadcasted_iota(jnp.int32, (EB, 1), 0)) >= E
    outs = (o0, o1, o2, o3)
    for i in range(L):
        v = (jnp.dot(z2, wt_ref[i, :, :], preferred_element_type=F32)
             + bt_ref[i:i + 1, :])
        outs[i][...] = jnp.where(pad, -1e4, v)


@jax.jit
def _edge_cs(eap, edge_w1, edge_b1, wt, bt):
    es = pl.BlockSpec((EB, H), lambda i: (i, 0))
    return pl.pallas_call(
        _edge_cs_body,
        grid=(E_PAD // EB,),
        in_specs=[
            pl.BlockSpec((EB, DE), lambda i: (i, 0)),
            pl.BlockSpec((DE, H), lambda i: (0, 0)),
            pl.BlockSpec((1, H), lambda i: (0, 0)),
            pl.BlockSpec((L, H, H), lambda i: (0, 0, 0)),
            pl.BlockSpec((L, H), lambda i: (0, 0)),
        ],
        out_specs=(es, es, es, es),
        out_shape=tuple(jax.ShapeDtypeStruct((E_PAD, H), F32)
                        for _ in range(L)),
    )(eap, edge_w1, edge_b1.reshape(1, -1), wt, bt)


def _npre_body(h_ref, wa_ref, wb_ref, a_ref, b_ref):
    h = h_ref[...]
    a_ref[...] = jnp.dot(h, wa_ref[...], preferred_element_type=F32)
    b_ref[...] = jnp.dot(h, wb_ref[...], preferred_element_type=F32)


@jax.jit
def _node_pre(h, wa, wb):
    ns = pl.BlockSpec((NB, H), lambda i: (i, 0))
    ws = pl.BlockSpec((H, H), lambda i: (0, 0))
    return pl.pallas_call(
        _npre_body,
        grid=(N // NB,),
        in_specs=[ns, ws, ws],
        out_specs=(ns, ns),
        out_shape=(jax.ShapeDtypeStruct((N, H), F32),
                   jax.ShapeDtypeStruct((N, H), F32)),
    )(h, wa, wb)


def _upd_body(h_ref, agg_ref, deg_ref, u1a, p_ref, q_ref, b1, u2, b2,
              g_ref, bt_ref, o_ref):
    h = h_ref[...]
    agg = agg_ref[0] + agg_ref[1]
    deg = deg_ref[0] + deg_ref[1]
    a = (jnp.dot(h, u1a[...], preferred_element_type=F32)
         + jnp.dot(agg, p_ref[...], preferred_element_type=F32)
         + deg * q_ref[...]
         + b1[...])
    u = jnp.dot(_silu(a), u2[...], preferred_element_type=F32) + b2[...]
    y = h + u
    mu = jnp.mean(y, axis=-1, keepdims=True)
    var = jnp.mean((y - mu) * (y - mu), axis=-1, keepdims=True)
    o_ref[...] = (y - mu) / jnp.sqrt(var + 1e-5) * g_ref[...] + bt_ref[...]


@jax.jit
def _update(h, agg2, deg2, u1, p, q, b1, u2, b2, g, bt):
    wspec = pl.BlockSpec((H, H), lambda i: (0, 0))
    bspec = pl.BlockSpec((1, H), lambda i: (0, 0))
    nspec = pl.BlockSpec((NB, H), lambda i: (i, 0))
    aspec = pl.BlockSpec((NC, NB, H), lambda i: (0, i, 0))
    return pl.pallas_call(
        _upd_body,
        grid=(N // NB,),
        in_specs=[nspec, aspec, aspec,
                  wspec, wspec, bspec, bspec, wspec, bspec, bspec, bspec],
        out_specs=nspec,
        out_shape=jax.ShapeDtypeStruct((N, H), F32),
    )(h, agg2, deg2, u1[:H], p, q.reshape(1, -1), b1.reshape(1, -1),
      u2, b2.reshape(1, -1), g.reshape(1, -1), bt.reshape(1, -1))


def _pool_body(h_ref, b_ref, sums_ref, cnts_ref, maxs_ref):
    i = pl.program_id(0)

    @pl.when(i == 0)
    def _init():
        sums_ref[...] = jnp.zeros((G, H), F32)
        cnts_ref[...] = jnp.zeros((G, H), F32)
        maxs_ref[...] = jnp.full((G, H), -1e30, F32)

    h = h_ref[...]
    b = b_ref[0, 0, :]
    gids = lax.broadcasted_iota(jnp.int32, (NB, G), 1)
    onehot = (b[:, None] == gids)
    oh_f = onehot.astype(F32)
    dn = (((0,), (0,)), ((), ()))
    sums_ref[...] += lax.dot_general(oh_f, h, dn, preferred_element_type=F32)
    cnts_ref[...] += lax.dot_general(oh_f, jnp.ones((NB, H), F32), dn,
                                     preferred_element_type=F32)
    parts = []
    for g in range(G):
        msk = b[:, None] == g
        parts.append(jnp.max(jnp.where(msk, h, -1e30), axis=0, keepdims=True))
    maxs_ref[...] = jnp.maximum(maxs_ref[...], jnp.concatenate(parts, axis=0))


@jax.jit
def _pool(h, batch3):
    gspec = pl.BlockSpec((G, H), lambda i: (0, 0))
    return pl.pallas_call(
        _pool_body,
        grid=(N // NB,),
        in_specs=[pl.BlockSpec((NB, H), lambda i: (i, 0)),
                  pl.BlockSpec((1, 1, NB), lambda i: (i, 0, 0))],
        out_specs=(gspec, gspec, gspec),
        out_shape=(jax.ShapeDtypeStruct((G, H), F32),
                   jax.ShapeDtypeStruct((G, H), F32),
                   jax.ShapeDtypeStruct((G, H), F32)),
    )(h, batch3)


def _head_body(sums_ref, cnts_ref, maxs_ref, w1a, w1b, b1, w2, b2,
               out_ref, gr_ref):
    cnts = cnts_ref[...]
    mean = sums_ref[...] / jnp.clip(cnts, 1.0, None)
    mx = jnp.where(cnts > 0, maxs_ref[...], 0.0)
    gr_ref[:, :H] = mean
    gr_ref[:, H:] = mx
    a = (jnp.dot(mean, w1a[...], preferred_element_type=F32)
         + jnp.dot(mx, w1b[...], preferred_element_type=F32)
         + b1[...])
    out_ref[...] = (jnp.dot(_silu(a), w2[...], preferred_element_type=F32)
                    + b2[...])


@jax.jit
def _head(sums, cnts, maxs, w1, b1, w2, b2):
    return pl.pallas_call(
        _head_body,
        out_shape=(jax.ShapeDtypeStruct((G, LAT), F32),
                   jax.ShapeDtypeStruct((G, 2 * H), F32)),
    )(sums, cnts, maxs, w1[:H], w1[H:], b1.reshape(1, -1), w2,
      b2.reshape(1, -1))


def kernel(x, edge_index, edge_attr, batch, node_w1, node_b1, node_w2,
           node_b2, edge_w1, edge_b1, edge_w2, edge_b2, msg_w1, msg_b1,
           msg_w2, msg_b2, upd_w1, upd_b1, upd_w2, upd_b2, ln_g, ln_b,
           out_w1, out_b1, out_w2, out_b2):
    srcp = jnp.pad(edge_index[0], (0, E_PAD - E)).reshape(-1, SPC * SCH)
    tgtu = jnp.pad(edge_index[1], (0, E_PAD - E)).reshape(-1, SCH)
    tgts64 = jnp.pad(edge_index[1], (0, E_PAD - E),
                     constant_values=N).reshape(-1, 64)
    eap = jnp.pad(edge_attr, ((0, E_PAD - E), (0, 0)))

    wt, btl, p_all, q_all = _weight_prep(msg_w1, msg_b1, msg_w2, msg_b2,
                                         upd_w1, edge_w2, edge_b2)
    h = _mlp2(x, node_w1, node_b1, node_w2, node_b2, NB)
    deg2 = _scatter_ones(tgts64)
    ces = _edge_cs(eap, edge_w1, edge_b1, wt, btl)

    for i in range(L):
        a_tab, b_tab = _node_pre(h, msg_w1[i][:H], msg_w1[i][H:2 * H])
        agg2 = _edge_sc(a_tab, b_tab, ces[i], srcp, tgtu)
        h = _update(h, agg2, deg2, upd_w1[i], p_all[i], q_all[i],
                    upd_b1[i], upd_w2[i], upd_b2[i], ln_g[i], ln_b[i])

    sums, cnts, maxs = _pool(h, batch.reshape(N // NB, 1, NB))
    return _head(sums, cnts, maxs, out_w1, out_b1, out_w2, out_b2)


# speedup vs baseline: 1.0036x; 1.0036x over previous
import jax
import jax.numpy as jnp
from jax import lax
from jax.experimental import pallas as pl
from jax.experimental.pallas import tpu as pltpu
from jax.experimental.pallas import tpu_sc as plsc

N = 10000
E = 160000
D = 256
DE = 16
H = 128
L = 4
LAT = 128
G = 64

NC = 2
NS = 16
NW = NC * NS
CHUNK = 128
E_PAD = 163840
N_PAD = 10240
ZPW = N_PAD // (NS * CHUNK)

SCH = 32
CN = E_PAD // (NW * SCH)
ZPS = N_PAD // (NS * SCH)
SPC = 128 // SCH

SPW = E_PAD // (NW * 64)
ZPSD = N_PAD // (NS * 64)

NBUF = 4

NB = 1000
EB = 2048
F32 = jnp.float32


def _silu(v):
    return v * jax.nn.sigmoid(v)


def _sc_edge_body(a_hbm, b_hbm, ce_hbm, srcp, tgtu, agg_out,
                  idx_s, idx_t, bufA, shared,
                  sem_l, sem_ga, sem_gb, sem_s, sem_w):
    cid = lax.axis_index("c")
    sid = lax.axis_index("s")
    wid = sid * NC + cid
    base = wid * CN

    def zrow(r, _):
        for q in range(H // 16):
            bufA[0, r, pl.ds(q * 16, 16)] = jnp.zeros((16,), F32)
        return _

    lax.fori_loop(0, SCH, zrow, None)
    for k in range(ZPS):
        pltpu.sync_copy(bufA.at[0],
                        shared.at[pl.ds((sid * ZPS + k) * SCH, SCH)])
    pltpu.sync_copy(srcp.at[pl.ds(wid * (CN // SPC), CN // SPC)], idx_s)
    pltpu.sync_copy(tgtu.at[pl.ds(base, CN)], idx_t)
    plsc.subcore_barrier()

    def s_wait(b):
        pltpu.make_async_copy(bufA.at[b], shared.at[pl.ds(0, SCH)],
                              sem_s.at[b]).wait()

    def st0(b, c):
        off = (base + c) * SCH
        pltpu.async_copy(ce_hbm.at[pl.ds(off, SCH)], bufA.at[b], sem_l.at[b])

    def st1(b, c):
        pltpu.make_async_copy(ce_hbm.at[pl.ds(0, SCH)], bufA.at[b],
                              sem_l.at[b]).wait()
        pltpu.async_copy(
            a_hbm.at[idx_s.at[c // SPC, pl.ds((c % SPC) * SCH, SCH)]],
            bufA.at[b], sem_ga.at[b], add=True)

    def st2(b, c):
        pltpu.make_async_copy(ce_hbm.at[pl.ds(0, SCH)], bufA.at[b],
                              sem_ga.at[b]).wait()
        pltpu.async_copy(b_hbm.at[idx_t.at[c]], bufA.at[b], sem_gb.at[b],
                         add=True)

    def st3(b, c):
        pltpu.make_async_copy(ce_hbm.at[pl.ds(0, SCH)], bufA.at[b],
                              sem_gb.at[b]).wait()

        def row(r, _):
            for q in range(H // 16):
                sl = pl.ds(q * 16, 16)
                z = bufA[b, r, sl]
                bufA[b, r, sl] = z / (1.0 + jnp.exp(-z))
            return _

        lax.fori_loop(0, SCH, row, None)
        pltpu.async_copy(bufA.at[b], shared.at[idx_t.at[c]], sem_s.at[b],
                         add=True)

    def step(s, _):
        for b in range(NBUF):
            t = s * NBUF + b

            @pl.when(t >= NBUF)
            def _():
                s_wait(b)
            st0(b, t)

            @pl.when(t >= 1)
            def _():
                st1((b + NBUF - 1) % NBUF, t - 1)

            @pl.when(t >= 2)
            def _():
                st2((b + NBUF - 2) % NBUF, t - 2)

            @pl.when(t >= 3)
            def _():
                st3((b + NBUF - 3) % NBUF, t - 3)
        return _

    lax.fori_loop(0, CN // NBUF, step, None)
    st1((CN - 1) % NBUF, CN - 1)
    st2((CN - 2) % NBUF, CN - 2)
    st2((CN - 1) % NBUF, CN - 1)
    for u in (CN - 3, CN - 2, CN - 1):
        st3(u % NBUF, u)
    for b in range(NBUF):
        s_wait(b)

    plsc.subcore_barrier()
    for k in range(ZPW):
        row = (sid * ZPW + k) * CHUNK
        pltpu.async_copy(shared.at[pl.ds(row, CHUNK)],
                         agg_out.at[cid, pl.ds(row, CHUNK)], sem_w)
    for k in range(ZPW):
        pltpu.make_async_copy(shared.at[pl.ds(0, CHUNK)],
                              agg_out.at[cid, pl.ds(0, CHUNK)], sem_w).wait()


@jax.jit
def _edge_sc(a_tab, b_tab, ce, srcp, tgtu):
    return pl.kernel(
        _sc_edge_body,
        out_type=jax.ShapeDtypeStruct((NC, N_PAD, H), F32),
        mesh=plsc.VectorSubcoreMesh(core_axis_name="c", subcore_axis_name="s"),
        scratch_types=[
            pltpu.VMEM((CN // SPC, SPC * SCH), jnp.int32),
            pltpu.VMEM((CN, SCH), jnp.int32),
            pltpu.VMEM((NBUF, SCH, H), F32),
            pltpu.VMEM_SHARED((N_PAD, H), F32),
            pltpu.SemaphoreType.DMA((NBUF,)),
            pltpu.SemaphoreType.DMA((NBUF,)),
            pltpu.SemaphoreType.DMA((NBUF,)),
            pltpu.SemaphoreType.DMA((NBUF,)),
            pltpu.SemaphoreType.DMA,
        ],
    )(a_tab, b_tab, ce, srcp, tgtu)


def _sc_deg_body(tgt_hbm, agg_out, idx_t, buf, shared, sem_a, sem_w):
    cid = lax.axis_index("c")
    sid = lax.axis_index("s")
    wid = sid * NC + cid
    base = wid * SPW

    def fill(v, row_buf):
        def zrow(r, _):
            for q in range(H // 16):
                buf[row_buf, r, pl.ds(q * 16, 16)] = jnp.full((16,), v, F32)
            return _
        lax.fori_loop(0, 64, zrow, None)

    fill(0.0, 0)
    for k in range(ZPSD):
        pltpu.sync_copy(buf.at[0], shared.at[pl.ds((sid * ZPSD + k) * 64, 64)])
    fill(1.0, 1)
    pltpu.sync_copy(tgt_hbm.at[pl.ds(base, SPW)], idx_t)
    plsc.subcore_barrier()

    def a_start(b, c):
        pltpu.async_copy(buf.at[1], shared.at[idx_t.at[c]], sem_a.at[b],
                         add=True)

    def a_wait(b):
        pltpu.make_async_copy(buf.at[1], shared.at[pl.ds(0, 64)],
                              sem_a.at[b]).wait()

    for b in range(NBUF):
        a_start(b, b)

    def step(s, _):
        for b in range(NBUF):
            c = s * NBUF + b

            @pl.when(c >= NBUF)
            def _():
                a_wait(b)
                a_start(b, c)
        return _

    lax.fori_loop(1, SPW // NBUF, step, None)
    for b in range(NBUF):
        a_wait(b)

    plsc.subcore_barrier()
    for k in range(ZPW):
        row = (sid * ZPW + k) * CHUNK
        pltpu.async_copy(shared.at[pl.ds(row, CHUNK)],
                         agg_out.at[cid, pl.ds(row, CHUNK)], sem_w)
    for k in range(ZPW):
        pltpu.make_async_copy(shared.at[pl.ds(0, CHUNK)],
                              agg_out.at[cid, pl.ds(0, CHUNK)], sem_w).wait()


@jax.jit
def _scatter_ones(tgt2s):
    return pl.kernel(
        _sc_deg_body,
        out_type=jax.ShapeDtypeStruct((NC, N_PAD, H), F32),
        mesh=plsc.VectorSubcoreMesh(core_axis_name="c", subcore_axis_name="s"),
        scratch_types=[
            pltpu.VMEM((SPW, 64), jnp.int32),
            pltpu.VMEM((2, 64, H), F32),
            pltpu.VMEM_SHARED((N_PAD, H), F32),
            pltpu.SemaphoreType.DMA((NBUF,)),
            pltpu.SemaphoreType.DMA,
        ],
    )(tgt2s)


def _mlp2_body(x_ref, w1_ref, b1_ref, w2_ref, b2_ref, o_ref):
    a = jnp.dot(x_ref[...], w1_ref[...], preferred_element_type=F32) + b1_ref[...]
    o_ref[...] = (jnp.dot(_silu(a), w2_ref[...], preferred_element_type=F32)
                  + b2_ref[...])


def _mlp2(x, w1, b1, w2, b2, blk):
    n, d = x.shape
    h1 = w1.shape[1]
    h2 = w2.shape[1]
    return pl.pallas_call(
        _mlp2_body,
        grid=(n // blk,),
        in_specs=[
            pl.BlockSpec((blk, d), lambda i: (i, 0)),
            pl.BlockSpec((d, h1), lambda i: (0, 0)),
            pl.BlockSpec((1, h1), lambda i: (0, 0)),
            pl.BlockSpec((h1, h2), lambda i: (0, 0)),
            pl.BlockSpec((1, h2), lambda i: (0, 0)),
        ],
        out_specs=pl.BlockSpec((blk, h2), lambda i: (i, 0)),
        out_shape=jax.ShapeDtypeStruct((n, h2), F32),
    )(x, w1, b1.reshape(1, -1), w2, b2.reshape(1, -1))


def _wprep_body(mw1_ref, mb1_ref, mw2_ref, mb2_ref, uw1_ref, ew2_ref,
                eb2_ref, wt_ref, bt_ref, p_ref, q_ref):
    for i in range(L):
        w1c = mw1_ref[i, 2 * H:, :]
        u1b = uw1_ref[i, H:, :]
        wt_ref[i, :, :] = jnp.dot(ew2_ref[...], w1c,
                                  preferred_element_type=F32)
        bt_ref[i:i + 1, :] = (jnp.dot(eb2_ref[...], w1c,
                                      preferred_element_type=F32)
                              + mb1_ref[i:i + 1, :])
        p_ref[i, :, :] = jnp.dot(mw2_ref[i, :, :], u1b,
                                 preferred_element_type=F32)
        q_ref[i:i + 1, :] = jnp.dot(mb2_ref[i:i + 1, :], u1b,
                                    preferred_element_type=F32)


@jax.jit
def _weight_prep(msg_w1, msg_b1, msg_w2, msg_b2, upd_w1, edge_w2, edge_b2):
    return pl.pallas_call(
        _wprep_body,
        out_shape=(jax.ShapeDtypeStruct((L, H, H), F32),
                   jax.ShapeDtypeStruct((L, H), F32),
                   jax.ShapeDtypeStruct((L, H, H), F32),
                   jax.ShapeDtypeStruct((L, H), F32)),
    )(msg_w1, msg_b1, msg_w2, msg_b2, upd_w1, edge_w2,
      edge_b2.reshape(1, -1))


def _edge_cs_body(ea_ref, w1_ref, b1_ref, wt_ref, bt_ref, o0, o1, o2, o3):
    z2 = _silu(jnp.dot(ea_ref[...], w1_ref[...], preferred_element_type=F32)
               + b1_ref[...])
    gbase = pl.program_id(0) * EB
    pad = (gbase + lax.broadcasted_iota(jnp.int32, (EB, 1), 0)) >= E
    outs = (o0, o1, o2, o3)
    for i in range(L):
        v = (jnp.dot(z2, wt_ref[i, :, :], preferred_element_type=F32)
             + bt_ref[i:i + 1, :])
        outs[i][...] = jnp.where(pad, -1e4, v)


@jax.jit
def _edge_cs(eap, edge_w1, edge_b1, wt, bt):
    es = pl.BlockSpec((EB, H), lambda i: (i, 0))
    return pl.pallas_call(
        _edge_cs_body,
        grid=(E_PAD // EB,),
        in_specs=[
            pl.BlockSpec((EB, DE), lambda i: (i, 0)),
            pl.BlockSpec((DE, H), lambda i: (0, 0)),
            pl.BlockSpec((1, H), lambda i: (0, 0)),
            pl.BlockSpec((L, H, H), lambda i: (0, 0, 0)),
            pl.BlockSpec((L, H), lambda i: (0, 0)),
        ],
        out_specs=(es, es, es, es),
        out_shape=tuple(jax.ShapeDtypeStruct((E_PAD, H), F32)
                        for _ in range(L)),
    )(eap, edge_w1, edge_b1.reshape(1, -1), wt, bt)


def _enc_fused_body(x_ref, w1_ref, b1_ref, w2_ref, b2_ref, wa_ref, wb_ref,
                    h_ref, a_ref, b_ref):
    a = (jnp.dot(x_ref[...], w1_ref[...], preferred_element_type=F32)
         + b1_ref[...])
    h = (jnp.dot(_silu(a), w2_ref[...], preferred_element_type=F32)
         + b2_ref[...])
    h_ref[...] = h
    a_ref[...] = jnp.dot(h, wa_ref[...], preferred_element_type=F32)
    b_ref[...] = jnp.dot(h, wb_ref[...], preferred_element_type=F32)


@jax.jit
def _enc_fused(x, w1, b1, w2, b2, wa, wb):
    ns = pl.BlockSpec((NB, H), lambda i: (i, 0))
    ws = pl.BlockSpec((H, H), lambda i: (0, 0))
    bs = pl.BlockSpec((1, H), lambda i: (0, 0))
    sd = jax.ShapeDtypeStruct((N, H), F32)
    return pl.pallas_call(
        _enc_fused_body,
        grid=(N // NB,),
        in_specs=[pl.BlockSpec((NB, D), lambda i: (i, 0)),
                  pl.BlockSpec((D, H), lambda i: (0, 0)),
                  bs, ws, bs, ws, ws],
        out_specs=(ns, ns, ns),
        out_shape=(sd, sd, sd),
    )(x, w1, b1.reshape(1, -1), w2, b2.reshape(1, -1), wa, wb)


def _ln_update(h, agg_ref, deg_ref, u1a, p_ref, q_ref, b1, u2, b2,
               g_ref, bt_ref):
    agg = agg_ref[0] + agg_ref[1]
    deg = deg_ref[0] + deg_ref[1]
    a = (jnp.dot(h, u1a[...], preferred_element_type=F32)
         + jnp.dot(agg, p_ref[...], preferred_element_type=F32)
         + deg * q_ref[...]
         + b1[...])
    u = jnp.dot(_silu(a), u2[...], preferred_element_type=F32) + b2[...]
    y = h + u
    mu = jnp.mean(y, axis=-1, keepdims=True)
    var = jnp.mean((y - mu) * (y - mu), axis=-1, keepdims=True)
    return (y - mu) / jnp.sqrt(var + 1e-5) * g_ref[...] + bt_ref[...]


def _upd_body(h_ref, agg_ref, deg_ref, u1a, p_ref, q_ref, b1, u2, b2,
              g_ref, bt_ref, o_ref):
    o_ref[...] = _ln_update(h_ref[...], agg_ref, deg_ref, u1a, p_ref, q_ref,
                            b1, u2, b2, g_ref, bt_ref)


def _updf_body(h_ref, agg_ref, deg_ref, u1a, p_ref, q_ref, b1, u2, b2,
               g_ref, bt_ref, wa_ref, wb_ref, o_ref, a_ref, b_ref):
    o = _ln_update(h_ref[...], agg_ref, deg_ref, u1a, p_ref, q_ref,
                   b1, u2, b2, g_ref, bt_ref)
    o_ref[...] = o
    a_ref[...] = jnp.dot(o, wa_ref[...], preferred_element_type=F32)
    b_ref[...] = jnp.dot(o, wb_ref[...], preferred_element_type=F32)


_WS = pl.BlockSpec((H, H), lambda i: (0, 0))
_BS = pl.BlockSpec((1, H), lambda i: (0, 0))
_NS = pl.BlockSpec((NB, H), lambda i: (i, 0))
_AS = pl.BlockSpec((NC, NB, H), lambda i: (0, i, 0))


@jax.jit
def _update(h, agg2, deg2, u1, p, q, b1, u2, b2, g, bt):
    return pl.pallas_call(
        _upd_body,
        grid=(N // NB,),
        in_specs=[_NS, _AS, _AS, _WS, _WS, _BS, _BS, _WS, _BS, _BS, _BS],
        out_specs=_NS,
        out_shape=jax.ShapeDtypeStruct((N, H), F32),
    )(h, agg2, deg2, u1[:H], p, q.reshape(1, -1), b1.reshape(1, -1),
      u2, b2.reshape(1, -1), g.reshape(1, -1), bt.reshape(1, -1))


@jax.jit
def _update_fused(h, agg2, deg2, u1, p, q, b1, u2, b2, g, bt, wa, wb):
    sd = jax.ShapeDtypeStruct((N, H), F32)
    return pl.pallas_call(
        _updf_body,
        grid=(N // NB,),
        in_specs=[_NS, _AS, _AS, _WS, _WS, _BS, _BS, _WS, _BS, _BS, _BS,
                  _WS, _WS],
        out_specs=(_NS, _NS, _NS),
        out_shape=(sd, sd, sd),
    )(h, agg2, deg2, u1[:H], p, q.reshape(1, -1), b1.reshape(1, -1),
      u2, b2.reshape(1, -1), g.reshape(1, -1), bt.reshape(1, -1), wa, wb)


def _pool_body(h_ref, b_ref, sums_ref, cnts_ref, maxs_ref):
    i = pl.program_id(0)

    @pl.when(i == 0)
    def _init():
        sums_ref[...] = jnp.zeros((G, H), F32)
        cnts_ref[...] = jnp.zeros((G, H), F32)
        maxs_ref[...] = jnp.full((G, H), -1e30, F32)

    h = h_ref[...]
    b = b_ref[0, 0, :]
    gids = lax.broadcasted_iota(jnp.int32, (NB, G), 1)
    onehot = (b[:, None] == gids)
    oh_f = onehot.astype(F32)
    dn = (((0,), (0,)), ((), ()))
    sums_ref[...] += lax.dot_general(oh_f, h, dn, preferred_element_type=F32)
    cnts_ref[...] += lax.dot_general(oh_f, jnp.ones((NB, H), F32), dn,
                                     preferred_element_type=F32)
    parts = []
    for g in range(G):
        msk = b[:, None] == g
        parts.append(jnp.max(jnp.where(msk, h, -1e30), axis=0, keepdims=True))
    maxs_ref[...] = jnp.maximum(maxs_ref[...], jnp.concatenate(parts, axis=0))


@jax.jit
def _pool(h, batch3):
    gspec = pl.BlockSpec((G, H), lambda i: (0, 0))
    return pl.pallas_call(
        _pool_body,
        grid=(N // NB,),
        in_specs=[pl.BlockSpec((NB, H), lambda i: (i, 0)),
                  pl.BlockSpec((1, 1, NB), lambda i: (i, 0, 0))],
        out_specs=(gspec, gspec, gspec),
        out_shape=(jax.ShapeDtypeStruct((G, H), F32),
                   jax.ShapeDtypeStruct((G, H), F32),
                   jax.ShapeDtypeStruct((G, H), F32)),
    )(h, batch3)


def _head_body(sums_ref, cnts_ref, maxs_ref, w1a, w1b, b1, w2, b2,
               out_ref, gr_ref):
    cnts = cnts_ref[...]
    mean = sums_ref[...] / jnp.clip(cnts, 1.0, None)
    mx = jnp.where(cnts > 0, maxs_ref[...], 0.0)
    gr_ref[:, :H] = mean
    gr_ref[:, H:] = mx
    a = (jnp.dot(mean, w1a[...], preferred_element_type=F32)
         + jnp.dot(mx, w1b[...], preferred_element_type=F32)
         + b1[...])
    out_ref[...] = (jnp.dot(_silu(a), w2[...], preferred_element_type=F32)
                    + b2[...])


@jax.jit
def _head(sums, cnts, maxs, w1, b1, w2, b2):
    return pl.pallas_call(
        _head_body,
        out_shape=(jax.ShapeDtypeStruct((G, LAT), F32),
                   jax.ShapeDtypeStruct((G, 2 * H), F32)),
    )(sums, cnts, maxs, w1[:H], w1[H:], b1.reshape(1, -1), w2,
      b2.reshape(1, -1))


def kernel(x, edge_index, edge_attr, batch, node_w1, node_b1, node_w2,
           node_b2, edge_w1, edge_b1, edge_w2, edge_b2, msg_w1, msg_b1,
           msg_w2, msg_b2, upd_w1, upd_b1, upd_w2, upd_b2, ln_g, ln_b,
           out_w1, out_b1, out_w2, out_b2):
    srcp = jnp.pad(edge_index[0], (0, E_PAD - E)).reshape(-1, SPC * SCH)
    tgtu = jnp.pad(edge_index[1], (0, E_PAD - E)).reshape(-1, SCH)
    tgts64 = jnp.pad(edge_index[1], (0, E_PAD - E),
                     constant_values=N).reshape(-1, 64)
    eap = jnp.pad(edge_attr, ((0, E_PAD - E), (0, 0)))

    wt, btl, p_all, q_all = _weight_prep(msg_w1, msg_b1, msg_w2, msg_b2,
                                         upd_w1, edge_w2, edge_b2)
    h, a_tab, b_tab = _enc_fused(x, node_w1, node_b1, node_w2, node_b2,
                                 msg_w1[0][:H], msg_w1[0][H:2 * H])
    deg2 = _scatter_ones(tgts64)
    ces = _edge_cs(eap, edge_w1, edge_b1, wt, btl)

    for i in range(L):
        agg2 = _edge_sc(a_tab, b_tab, ces[i], srcp, tgtu)
        args = (h, agg2, deg2, upd_w1[i], p_all[i], q_all[i],
                upd_b1[i], upd_w2[i], upd_b2[i], ln_g[i], ln_b[i])
        if i < L - 1:
            h, a_tab, b_tab = _update_fused(*args, msg_w1[i + 1][:H],
                                            msg_w1[i + 1][H:2 * H])
        else:
            h = _update(*args)

    sums, cnts, maxs = _pool(h, batch.reshape(N // NB, 1, NB))
    return _head(sums, cnts, maxs, out_w1, out_b1, out_w2, out_b2)


# speedup vs baseline: 1.0258x; 1.0220x over previous
import jax
import jax.numpy as jnp
from jax import lax
from jax.experimental import pallas as pl
from jax.experimental.pallas import tpu as pltpu
from jax.experimental.pallas import tpu_sc as plsc

N = 10000
E = 160000
D = 256
DE = 16
H = 128
L = 4
LAT = 128
G = 64

NC = 2
NS = 16
NW = NC * NS
CHUNK = 128
E_PAD = 163840
N_PAD = 10240
ZPW = N_PAD // (NS * CHUNK)

SCH = 32
CN = E_PAD // (NW * SCH)
ZPS = N_PAD // (NS * SCH)
SPC = 128 // SCH

SPW = E_PAD // (NW * 64)
ZPSD = N_PAD // (NS * 64)

NBUF = 4

NB = 1000
EB = 2048
F32 = jnp.float32


def _silu(v):
    return v * jax.nn.sigmoid(v)


def _sc_edge_body(a_hbm, b_hbm, ce_hbm, srcp, tgtu, agg_out,
                  idx_s, idx_t, bufA, shared,
                  sem_l, sem_ga, sem_gb, sem_s, sem_w):
    cid = lax.axis_index("c")
    sid = lax.axis_index("s")
    wid = sid * NC + cid
    base = wid * CN

    def zrow(r, _):
        for q in range(H // 16):
            bufA[0, r, pl.ds(q * 16, 16)] = jnp.zeros((16,), F32)
        return _

    lax.fori_loop(0, SCH, zrow, None)
    for k in range(ZPS):
        pltpu.sync_copy(bufA.at[0],
                        shared.at[pl.ds((sid * ZPS + k) * SCH, SCH)])
    pltpu.sync_copy(srcp.at[pl.ds(wid * (CN // SPC), CN // SPC)], idx_s)
    pltpu.sync_copy(tgtu.at[pl.ds(base, CN)], idx_t)
    plsc.subcore_barrier()

    def s_wait(b):
        pltpu.make_async_copy(bufA.at[b], shared.at[pl.ds(0, SCH)],
                              sem_s.at[b]).wait()

    def st0(b, c):
        off = (base + c) * SCH
        pltpu.async_copy(ce_hbm.at[pl.ds(off, SCH)], bufA.at[b], sem_l.at[b])

    def st1(b, c):
        pltpu.make_async_copy(ce_hbm.at[pl.ds(0, SCH)], bufA.at[b],
                              sem_l.at[b]).wait()
        pltpu.async_copy(
            a_hbm.at[idx_s.at[c // SPC, pl.ds((c % SPC) * SCH, SCH)]],
            bufA.at[b], sem_ga.at[b], add=True)

    def st2(b, c):
        pltpu.make_async_copy(ce_hbm.at[pl.ds(0, SCH)], bufA.at[b],
                              sem_ga.at[b]).wait()
        pltpu.async_copy(b_hbm.at[idx_t.at[c]], bufA.at[b], sem_gb.at[b],
                         add=True)

    def st3(b, c):
        pltpu.make_async_copy(ce_hbm.at[pl.ds(0, SCH)], bufA.at[b],
                              sem_gb.at[b]).wait()

        def row(r, _):
            for q in range(H // 16):
                sl = pl.ds(q * 16, 16)
                z = bufA[b, r, sl]
                bufA[b, r, sl] = z / (1.0 + jnp.exp(-z))
            return _

        lax.fori_loop(0, SCH, row, None)
        pltpu.async_copy(bufA.at[b], shared.at[idx_t.at[c]], sem_s.at[b],
                         add=True)

    def step(s, _):
        for b in range(NBUF):
            t = s * NBUF + b

            @pl.when(t >= NBUF)
            def _():
                s_wait(b)
            st0(b, t)

            @pl.when(t >= 1)
            def _():
                st1((b + NBUF - 1) % NBUF, t - 1)

            @pl.when(t >= 2)
            def _():
                st2((b + NBUF - 2) % NBUF, t - 2)

            @pl.when(t >= 3)
            def _():
                st3((b + NBUF - 3) % NBUF, t - 3)
        return _

    lax.fori_loop(0, CN // NBUF, step, None)
    st1((CN - 1) % NBUF, CN - 1)
    st2((CN - 2) % NBUF, CN - 2)
    st2((CN - 1) % NBUF, CN - 1)
    for u in (CN - 3, CN - 2, CN - 1):
        st3(u % NBUF, u)
    for b in range(NBUF):
        s_wait(b)

    plsc.subcore_barrier()
    for k in range(ZPW):
        row = (sid * ZPW + k) * CHUNK
        pltpu.async_copy(shared.at[pl.ds(row, CHUNK)],
                         agg_out.at[cid, pl.ds(row, CHUNK)], sem_w)
    for k in range(ZPW):
        pltpu.make_async_copy(shared.at[pl.ds(0, CHUNK)],
                              agg_out.at[cid, pl.ds(0, CHUNK)], sem_w).wait()


@jax.jit
def _edge_sc(a_tab, b_tab, ce, srcp, tgtu):
    return pl.kernel(
        _sc_edge_body,
        out_type=jax.ShapeDtypeStruct((NC, N_PAD, H), F32),
        mesh=plsc.VectorSubcoreMesh(core_axis_name="c", subcore_axis_name="s"),
        scratch_types=[
            pltpu.VMEM((CN // SPC, SPC * SCH), jnp.int32),
            pltpu.VMEM((CN, SCH), jnp.int32),
            pltpu.VMEM((NBUF, SCH, H), F32),
            pltpu.VMEM_SHARED((N_PAD, H), F32),
            pltpu.SemaphoreType.DMA((NBUF,)),
            pltpu.SemaphoreType.DMA((NBUF,)),
            pltpu.SemaphoreType.DMA((NBUF,)),
            pltpu.SemaphoreType.DMA((NBUF,)),
            pltpu.SemaphoreType.DMA,
        ],
    )(a_tab, b_tab, ce, srcp, tgtu)


def _sc_deg_body(tgt_hbm, agg_out, idx_t, buf, shared, sem_a, sem_w):
    cid = lax.axis_index("c")
    sid = lax.axis_index("s")
    wid = sid * NC + cid
    base = wid * SPW

    def fill(v, row_buf):
        def zrow(r, _):
            for q in range(H // 16):
                buf[row_buf, r, pl.ds(q * 16, 16)] = jnp.full((16,), v, F32)
            return _
        lax.fori_loop(0, 64, zrow, None)

    fill(0.0, 0)
    for k in range(ZPSD):
        pltpu.sync_copy(buf.at[0], shared.at[pl.ds((sid * ZPSD + k) * 64, 64)])
    fill(1.0, 1)
    pltpu.sync_copy(tgt_hbm.at[pl.ds(base, SPW)], idx_t)
    plsc.subcore_barrier()

    def a_start(b, c):
        pltpu.async_copy(buf.at[1], shared.at[idx_t.at[c]], sem_a.at[b],
                         add=True)

    def a_wait(b):
        pltpu.make_async_copy(buf.at[1], shared.at[pl.ds(0, 64)],
                              sem_a.at[b]).wait()

    for b in range(NBUF):
        a_start(b, b)

    def step(s, _):
        for b in range(NBUF):
            c = s * NBUF + b

            @pl.when(c >= NBUF)
            def _():
                a_wait(b)
                a_start(b, c)
        return _

    lax.fori_loop(1, SPW // NBUF, step, None)
    for b in range(NBUF):
        a_wait(b)

    plsc.subcore_barrier()
    for k in range(ZPW):
        row = (sid * ZPW + k) * CHUNK
        pltpu.async_copy(shared.at[pl.ds(row, CHUNK)],
                         agg_out.at[cid, pl.ds(row, CHUNK)], sem_w)
    for k in range(ZPW):
        pltpu.make_async_copy(shared.at[pl.ds(0, CHUNK)],
                              agg_out.at[cid, pl.ds(0, CHUNK)], sem_w).wait()


@jax.jit
def _scatter_ones(tgt2s):
    return pl.kernel(
        _sc_deg_body,
        out_type=jax.ShapeDtypeStruct((NC, N_PAD, H), F32),
        mesh=plsc.VectorSubcoreMesh(core_axis_name="c", subcore_axis_name="s"),
        scratch_types=[
            pltpu.VMEM((SPW, 64), jnp.int32),
            pltpu.VMEM((2, 64, H), F32),
            pltpu.VMEM_SHARED((N_PAD, H), F32),
            pltpu.SemaphoreType.DMA((NBUF,)),
            pltpu.SemaphoreType.DMA,
        ],
    )(tgt2s)


def _mlp2_body(x_ref, w1_ref, b1_ref, w2_ref, b2_ref, o_ref):
    a = jnp.dot(x_ref[...], w1_ref[...], preferred_element_type=F32) + b1_ref[...]
    o_ref[...] = (jnp.dot(_silu(a), w2_ref[...], preferred_element_type=F32)
                  + b2_ref[...])


def _mlp2(x, w1, b1, w2, b2, blk):
    n, d = x.shape
    h1 = w1.shape[1]
    h2 = w2.shape[1]
    return pl.pallas_call(
        _mlp2_body,
        grid=(n // blk,),
        in_specs=[
            pl.BlockSpec((blk, d), lambda i: (i, 0)),
            pl.BlockSpec((d, h1), lambda i: (0, 0)),
            pl.BlockSpec((1, h1), lambda i: (0, 0)),
            pl.BlockSpec((h1, h2), lambda i: (0, 0)),
            pl.BlockSpec((1, h2), lambda i: (0, 0)),
        ],
        out_specs=pl.BlockSpec((blk, h2), lambda i: (i, 0)),
        out_shape=jax.ShapeDtypeStruct((n, h2), F32),
    )(x, w1, b1.reshape(1, -1), w2, b2.reshape(1, -1))


def _wprep_body(mw1_ref, mb1_ref, mw2_ref, mb2_ref, uw1_ref, ew2_ref,
                eb2_ref, wt_ref, bt_ref, p_ref, q_ref):
    for i in range(L):
        w1c = mw1_ref[i, 2 * H:, :]
        u1b = uw1_ref[i, H:, :]
        wt_ref[i, :, :] = jnp.dot(ew2_ref[...], w1c,
                                  preferred_element_type=F32)
        bt_ref[i:i + 1, :] = (jnp.dot(eb2_ref[...], w1c,
                                      preferred_element_type=F32)
                              + mb1_ref[i:i + 1, :])
        p_ref[i, :, :] = jnp.dot(mw2_ref[i, :, :], u1b,
                                 preferred_element_type=F32)
        q_ref[i:i + 1, :] = jnp.dot(mb2_ref[i:i + 1, :], u1b,
                                    preferred_element_type=F32)


@jax.jit
def _weight_prep(msg_w1, msg_b1, msg_w2, msg_b2, upd_w1, edge_w2, edge_b2):
    return pl.pallas_call(
        _wprep_body,
        out_shape=(jax.ShapeDtypeStruct((L, H, H), F32),
                   jax.ShapeDtypeStruct((L, H), F32),
                   jax.ShapeDtypeStruct((L, H, H), F32),
                   jax.ShapeDtypeStruct((L, H), F32)),
    )(msg_w1, msg_b1, msg_w2, msg_b2, upd_w1, edge_w2,
      edge_b2.reshape(1, -1))


def _edge_cs_body(ea_ref, w1_ref, b1_ref, wt_ref, bt_ref, o0, o1, o2, o3):
    z2 = _silu(jnp.dot(ea_ref[...], w1_ref[...], preferred_element_type=F32)
               + b1_ref[...])
    gbase = pl.program_id(0) * EB
    pad = (gbase + lax.broadcasted_iota(jnp.int32, (EB, 1), 0)) >= E
    outs = (o0, o1, o2, o3)
    for i in range(L):
        v = (jnp.dot(z2, wt_ref[i, :, :], preferred_element_type=F32)
             + bt_ref[i:i + 1, :])
        outs[i][...] = jnp.where(pad, -1e4, v)


@jax.jit
def _edge_cs(eap, edge_w1, edge_b1, wt, bt):
    es = pl.BlockSpec((EB, H), lambda i: (i, 0))
    return pl.pallas_call(
        _edge_cs_body,
        grid=(E_PAD // EB,),
        in_specs=[
            pl.BlockSpec((EB, DE), lambda i: (i, 0)),
            pl.BlockSpec((DE, H), lambda i: (0, 0)),
            pl.BlockSpec((1, H), lambda i: (0, 0)),
            pl.BlockSpec((L, H, H), lambda i: (0, 0, 0)),
            pl.BlockSpec((L, H), lambda i: (0, 0)),
        ],
        out_specs=(es, es, es, es),
        out_shape=tuple(jax.ShapeDtypeStruct((E_PAD, H), F32)
                        for _ in range(L)),
    )(eap, edge_w1, edge_b1.reshape(1, -1), wt, bt)


def _npre_body(h_ref, wa_ref, wb_ref, a_ref, b_ref):
    h = h_ref[...]
    a_ref[...] = jnp.dot(h, wa_ref[...], preferred_element_type=F32)
    b_ref[...] = jnp.dot(h, wb_ref[...], preferred_element_type=F32)


@jax.jit
def _node_pre(h, wa, wb):
    ns = pl.BlockSpec((NB, H), lambda i: (i, 0))
    ws = pl.BlockSpec((H, H), lambda i: (0, 0))
    return pl.pallas_call(
        _npre_body,
        grid=(N // NB,),
        in_specs=[ns, ws, ws],
        out_specs=(ns, ns),
        out_shape=(jax.ShapeDtypeStruct((N, H), F32),
                   jax.ShapeDtypeStruct((N, H), F32)),
    )(h, wa, wb)


def _ln_update(h, agg_ref, deg_ref, u1a, p_ref, q_ref, b1, u2, b2,
               g_ref, bt_ref):
    agg = agg_ref[0] + agg_ref[1]
    deg = deg_ref[0] + deg_ref[1]
    a = (jnp.dot(h, u1a[...], preferred_element_type=F32)
         + jnp.dot(agg, p_ref[...], preferred_element_type=F32)
         + deg * q_ref[...]
         + b1[...])
    u = jnp.dot(_silu(a), u2[...], preferred_element_type=F32) + b2[...]
    y = h + u
    mu = jnp.mean(y, axis=-1, keepdims=True)
    var = jnp.mean((y - mu) * (y - mu), axis=-1, keepdims=True)
    return (y - mu) / jnp.sqrt(var + 1e-5) * g_ref[...] + bt_ref[...]


def _upd_body(h_ref, agg_ref, deg_ref, u1a, p_ref, q_ref, b1, u2, b2,
              g_ref, bt_ref, o_ref):
    o_ref[...] = _ln_update(h_ref[...], agg_ref, deg_ref, u1a, p_ref, q_ref,
                            b1, u2, b2, g_ref, bt_ref)


_WS = pl.BlockSpec((H, H), lambda i: (0, 0))
_BS = pl.BlockSpec((1, H), lambda i: (0, 0))
_NS = pl.BlockSpec((NB, H), lambda i: (i, 0))
_AS = pl.BlockSpec((NC, NB, H), lambda i: (0, i, 0))


@jax.jit
def _update(h, agg2, deg2, u1, p, q, b1, u2, b2, g, bt):
    return pl.pallas_call(
        _upd_body,
        grid=(N // NB,),
        in_specs=[_NS, _AS, _AS, _WS, _WS, _BS, _BS, _WS, _BS, _BS, _BS],
        out_specs=_NS,
        out_shape=jax.ShapeDtypeStruct((N, H), F32),
    )(h, agg2, deg2, u1[:H], p, q.reshape(1, -1), b1.reshape(1, -1),
      u2, b2.reshape(1, -1), g.reshape(1, -1), bt.reshape(1, -1))


def _pool_body(h_ref, b_ref, sums_ref, cnts_ref, maxs_ref):
    i = pl.program_id(0)

    @pl.when(i == 0)
    def _init():
        sums_ref[...] = jnp.zeros((G, H), F32)
        cnts_ref[...] = jnp.zeros((G, H), F32)
        maxs_ref[...] = jnp.full((G, H), -1e30, F32)

    h = h_ref[...]
    b = b_ref[0, 0, :]
    gids = lax.broadcasted_iota(jnp.int32, (NB, G), 1)
    onehot = (b[:, None] == gids)
    oh_f = onehot.astype(F32)
    dn = (((0,), (0,)), ((), ()))
    sums_ref[...] += lax.dot_general(oh_f, h, dn, preferred_element_type=F32)
    cnts_ref[...] += lax.dot_general(oh_f, jnp.ones((NB, H), F32), dn,
                                     preferred_element_type=F32)
    parts = []
    for g in range(G):
        msk = b[:, None] == g
        parts.append(jnp.max(jnp.where(msk, h, -1e30), axis=0, keepdims=True))
    maxs_ref[...] = jnp.maximum(maxs_ref[...], jnp.concatenate(parts, axis=0))


@jax.jit
def _pool(h, batch3):
    gspec = pl.BlockSpec((G, H), lambda i: (0, 0))
    return pl.pallas_call(
        _pool_body,
        grid=(N // NB,),
        in_specs=[pl.BlockSpec((NB, H), lambda i: (i, 0)),
                  pl.BlockSpec((1, 1, NB), lambda i: (i, 0, 0))],
        out_specs=(gspec, gspec, gspec),
        out_shape=(jax.ShapeDtypeStruct((G, H), F32),
                   jax.ShapeDtypeStruct((G, H), F32),
                   jax.ShapeDtypeStruct((G, H), F32)),
    )(h, batch3)


def _head_body(sums_ref, cnts_ref, maxs_ref, w1a, w1b, b1, w2, b2,
               out_ref, gr_ref):
    cnts = cnts_ref[...]
    mean = sums_ref[...] / jnp.clip(cnts, 1.0, None)
    mx = jnp.where(cnts > 0, maxs_ref[...], 0.0)
    gr_ref[:, :H] = mean
    gr_ref[:, H:] = mx
    a = (jnp.dot(mean, w1a[...], preferred_element_type=F32)
         + jnp.dot(mx, w1b[...], preferred_element_type=F32)
         + b1[...])
    out_ref[...] = (jnp.dot(_silu(a), w2[...], preferred_element_type=F32)
                    + b2[...])


@jax.jit
def _head(sums, cnts, maxs, w1, b1, w2, b2):
    return pl.pallas_call(
        _head_body,
        out_shape=(jax.ShapeDtypeStruct((G, LAT), F32),
                   jax.ShapeDtypeStruct((G, 2 * H), F32)),
    )(sums, cnts, maxs, w1[:H], w1[H:], b1.reshape(1, -1), w2,
      b2.reshape(1, -1))


def kernel(x, edge_index, edge_attr, batch, node_w1, node_b1, node_w2,
           node_b2, edge_w1, edge_b1, edge_w2, edge_b2, msg_w1, msg_b1,
           msg_w2, msg_b2, upd_w1, upd_b1, upd_w2, upd_b2, ln_g, ln_b,
           out_w1, out_b1, out_w2, out_b2):
    srcp = jnp.pad(edge_index[0], (0, E_PAD - E)).reshape(-1, SPC * SCH)
    tgtu = jnp.pad(edge_index[1], (0, E_PAD - E)).reshape(-1, SCH)
    tgts64 = jnp.pad(edge_index[1], (0, E_PAD - E),
                     constant_values=N).reshape(-1, 64)
    eap = jnp.pad(edge_attr, ((0, E_PAD - E), (0, 0)))

    wt, btl, p_all, q_all = _weight_prep(msg_w1, msg_b1, msg_w2, msg_b2,
                                         upd_w1, edge_w2, edge_b2)
    h = _mlp2(x, node_w1, node_b1, node_w2, node_b2, NB)
    deg2 = _scatter_ones(tgts64)
    ces = _edge_cs(eap, edge_w1, edge_b1, wt, btl)

    for i in range(L):
        a_tab, b_tab = _node_pre(h, msg_w1[i][:H], msg_w1[i][H:2 * H])
        agg2 = _edge_sc(a_tab, b_tab, ces[i], srcp, tgtu)
        h = _update(h, agg2, deg2, upd_w1[i], p_all[i], q_all[i],
                    upd_b1[i], upd_w2[i], upd_b2[i], ln_g[i], ln_b[i])

    sums, cnts, maxs = _pool(h, batch.reshape(N // NB, 1, NB))
    return _head(sums, cnts, maxs, out_w1, out_b1, out_w2, out_b2)


# speedup vs baseline: 1.0266x; 1.0008x over previous
import jax
import jax.numpy as jnp
from jax import lax
from jax.experimental import pallas as pl
from jax.experimental.pallas import tpu as pltpu
from jax.experimental.pallas import tpu_sc as plsc

N = 10000
E = 160000
D = 256
DE = 16
H = 128
L = 4
LAT = 128
G = 64

NC = 2
NS = 16
NW = NC * NS
CHUNK = 128
E_PAD = 163840
N_PAD = 10240
ZPW = N_PAD // (NS * CHUNK)

SCH = 32
CN = E_PAD // (NW * SCH)
ZPS = N_PAD // (NS * SCH)
SPC = 128 // SCH

SPW = E_PAD // (NW * 64)
ZPSD = N_PAD // (NS * 64)

NBUF = 5

NB = 1000
EB = 2048
F32 = jnp.float32


def _silu(v):
    return v * jax.nn.sigmoid(v)


def _sc_edge_body(a_hbm, b_hbm, ce_hbm, srcp, tgtu, agg_out,
                  idx_s, idx_t, bufA, shared,
                  sem_l, sem_ga, sem_gb, sem_s, sem_w):
    cid = lax.axis_index("c")
    sid = lax.axis_index("s")
    wid = sid * NC + cid
    base = wid * CN

    def zrow(r, _):
        for q in range(H // 16):
            bufA[0, r, pl.ds(q * 16, 16)] = jnp.zeros((16,), F32)
        return _

    lax.fori_loop(0, SCH, zrow, None)
    for k in range(ZPS):
        pltpu.sync_copy(bufA.at[0],
                        shared.at[pl.ds((sid * ZPS + k) * SCH, SCH)])
    pltpu.sync_copy(srcp.at[pl.ds(wid * (CN // SPC), CN // SPC)], idx_s)
    pltpu.sync_copy(tgtu.at[pl.ds(base, CN)], idx_t)
    plsc.subcore_barrier()

    def s_wait(b):
        pltpu.make_async_copy(bufA.at[b], shared.at[pl.ds(0, SCH)],
                              sem_s.at[b]).wait()

    def st0(b, c):
        off = (base + c) * SCH
        pltpu.async_copy(ce_hbm.at[pl.ds(off, SCH)], bufA.at[b], sem_l.at[b])

    def st1(b, c):
        pltpu.make_async_copy(ce_hbm.at[pl.ds(0, SCH)], bufA.at[b],
                              sem_l.at[b]).wait()
        pltpu.async_copy(
            a_hbm.at[idx_s.at[c // SPC, pl.ds((c % SPC) * SCH, SCH)]],
            bufA.at[b], sem_ga.at[b], add=True)

    def st2(b, c):
        pltpu.make_async_copy(ce_hbm.at[pl.ds(0, SCH)], bufA.at[b],
                              sem_ga.at[b]).wait()
        pltpu.async_copy(b_hbm.at[idx_t.at[c]], bufA.at[b], sem_gb.at[b],
                         add=True)

    def st3(b, c):
        pltpu.make_async_copy(ce_hbm.at[pl.ds(0, SCH)], bufA.at[b],
                              sem_gb.at[b]).wait()

        def row(r, _):
            for q in range(H // 16):
                sl = pl.ds(q * 16, 16)
                z = bufA[b, r, sl]
                bufA[b, r, sl] = z / (1.0 + jnp.exp(-z))
            return _

        lax.fori_loop(0, SCH, row, None)
        pltpu.async_copy(bufA.at[b], shared.at[idx_t.at[c]], sem_s.at[b],
                         add=True)

    def step(s, _):
        for b in range(NBUF):
            t = s * NBUF + b

            @pl.when(t >= NBUF)
            def _():
                s_wait(b)
            st0(b, t)

            @pl.when(t >= 1)
            def _():
                st1((b + NBUF - 1) % NBUF, t - 1)

            @pl.when(t >= 2)
            def _():
                st2((b + NBUF - 2) % NBUF, t - 2)

            @pl.when(t >= 3)
            def _():
                st3((b + NBUF - 3) % NBUF, t - 3)
        return _

    lax.fori_loop(0, CN // NBUF, step, None)
    st1((CN - 1) % NBUF, CN - 1)
    st2((CN - 2) % NBUF, CN - 2)
    st2((CN - 1) % NBUF, CN - 1)
    for u in (CN - 3, CN - 2, CN - 1):
        st3(u % NBUF, u)
    for b in range(NBUF):
        s_wait(b)

    plsc.subcore_barrier()
    for k in range(ZPW):
        row = (sid * ZPW + k) * CHUNK
        pltpu.async_copy(shared.at[pl.ds(row, CHUNK)],
                         agg_out.at[cid, pl.ds(row, CHUNK)], sem_w)
    for k in range(ZPW):
        pltpu.make_async_copy(shared.at[pl.ds(0, CHUNK)],
                              agg_out.at[cid, pl.ds(0, CHUNK)], sem_w).wait()


@jax.jit
def _edge_sc(a_tab, b_tab, ce, srcp, tgtu):
    return pl.kernel(
        _sc_edge_body,
        out_type=jax.ShapeDtypeStruct((NC, N_PAD, H), F32),
        mesh=plsc.VectorSubcoreMesh(core_axis_name="c", subcore_axis_name="s"),
        scratch_types=[
            pltpu.VMEM((CN // SPC, SPC * SCH), jnp.int32),
            pltpu.VMEM((CN, SCH), jnp.int32),
            pltpu.VMEM((NBUF, SCH, H), F32),
            pltpu.VMEM_SHARED((N_PAD, H), F32),
            pltpu.SemaphoreType.DMA((NBUF,)),
            pltpu.SemaphoreType.DMA((NBUF,)),
            pltpu.SemaphoreType.DMA((NBUF,)),
            pltpu.SemaphoreType.DMA((NBUF,)),
            pltpu.SemaphoreType.DMA,
        ],
    )(a_tab, b_tab, ce, srcp, tgtu)


def _sc_deg_body(tgt_hbm, agg_out, idx_t, buf, shared, sem_a, sem_w):
    cid = lax.axis_index("c")
    sid = lax.axis_index("s")
    wid = sid * NC + cid
    base = wid * SPW

    def fill(v, row_buf):
        def zrow(r, _):
            for q in range(H // 16):
                buf[row_buf, r, pl.ds(q * 16, 16)] = jnp.full((16,), v, F32)
            return _
        lax.fori_loop(0, 64, zrow, None)

    fill(0.0, 0)
    for k in range(ZPSD):
        pltpu.sync_copy(buf.at[0], shared.at[pl.ds((sid * ZPSD + k) * 64, 64)])
    fill(1.0, 1)
    pltpu.sync_copy(tgt_hbm.at[pl.ds(base, SPW)], idx_t)
    plsc.subcore_barrier()

    def a_start(b, c):
        pltpu.async_copy(buf.at[1], shared.at[idx_t.at[c]], sem_a.at[b],
                         add=True)

    def a_wait(b):
        pltpu.make_async_copy(buf.at[1], shared.at[pl.ds(0, 64)],
                              sem_a.at[b]).wait()

    for b in range(NBUF):
        a_start(b, b)

    def step(s, _):
        for b in range(NBUF):
            c = s * NBUF + b

            @pl.when(c >= NBUF)
            def _():
                a_wait(b)
                a_start(b, c)
        return _

    lax.fori_loop(1, SPW // NBUF, step, None)
    for b in range(NBUF):
        a_wait(b)

    plsc.subcore_barrier()
    for k in range(ZPW):
        row = (sid * ZPW + k) * CHUNK
        pltpu.async_copy(shared.at[pl.ds(row, CHUNK)],
                         agg_out.at[cid, pl.ds(row, CHUNK)], sem_w)
    for k in range(ZPW):
        pltpu.make_async_copy(shared.at[pl.ds(0, CHUNK)],
                              agg_out.at[cid, pl.ds(0, CHUNK)], sem_w).wait()


@jax.jit
def _scatter_ones(tgt2s):
    return pl.kernel(
        _sc_deg_body,
        out_type=jax.ShapeDtypeStruct((NC, N_PAD, H), F32),
        mesh=plsc.VectorSubcoreMesh(core_axis_name="c", subcore_axis_name="s"),
        scratch_types=[
            pltpu.VMEM((SPW, 64), jnp.int32),
            pltpu.VMEM((2, 64, H), F32),
            pltpu.VMEM_SHARED((N_PAD, H), F32),
            pltpu.SemaphoreType.DMA((NBUF,)),
            pltpu.SemaphoreType.DMA,
        ],
    )(tgt2s)


def _mlp2_body(x_ref, w1_ref, b1_ref, w2_ref, b2_ref, o_ref):
    a = jnp.dot(x_ref[...], w1_ref[...], preferred_element_type=F32) + b1_ref[...]
    o_ref[...] = (jnp.dot(_silu(a), w2_ref[...], preferred_element_type=F32)
                  + b2_ref[...])


def _mlp2(x, w1, b1, w2, b2, blk):
    n, d = x.shape
    h1 = w1.shape[1]
    h2 = w2.shape[1]
    return pl.pallas_call(
        _mlp2_body,
        grid=(n // blk,),
        in_specs=[
            pl.BlockSpec((blk, d), lambda i: (i, 0)),
            pl.BlockSpec((d, h1), lambda i: (0, 0)),
            pl.BlockSpec((1, h1), lambda i: (0, 0)),
            pl.BlockSpec((h1, h2), lambda i: (0, 0)),
            pl.BlockSpec((1, h2), lambda i: (0, 0)),
        ],
        out_specs=pl.BlockSpec((blk, h2), lambda i: (i, 0)),
        out_shape=jax.ShapeDtypeStruct((n, h2), F32),
    )(x, w1, b1.reshape(1, -1), w2, b2.reshape(1, -1))


def _wprep_body(mw1_ref, mb1_ref, mw2_ref, mb2_ref, uw1_ref, ew2_ref,
                eb2_ref, wt_ref, bt_ref, p_ref, q_ref):
    for i in range(L):
        w1c = mw1_ref[i, 2 * H:, :]
        u1b = uw1_ref[i, H:, :]
        wt_ref[i, :, :] = jnp.dot(ew2_ref[...], w1c,
                                  preferred_element_type=F32)
        bt_ref[i:i + 1, :] = (jnp.dot(eb2_ref[...], w1c,
                                      preferred_element_type=F32)
                              + mb1_ref[i:i + 1, :])
        p_ref[i, :, :] = jnp.dot(mw2_ref[i, :, :], u1b,
                                 preferred_element_type=F32)
        q_ref[i:i + 1, :] = jnp.dot(mb2_ref[i:i + 1, :], u1b,
                                    preferred_element_type=F32)


@jax.jit
def _weight_prep(msg_w1, msg_b1, msg_w2, msg_b2, upd_w1, edge_w2, edge_b2):
    return pl.pallas_call(
        _wprep_body,
        out_shape=(jax.ShapeDtypeStruct((L, H, H), F32),
                   jax.ShapeDtypeStruct((L, H), F32),
                   jax.ShapeDtypeStruct((L, H, H), F32),
                   jax.ShapeDtypeStruct((L, H), F32)),
    )(msg_w1, msg_b1, msg_w2, msg_b2, upd_w1, edge_w2,
      edge_b2.reshape(1, -1))


def _edge_cs_body(ea_ref, w1_ref, b1_ref, wt_ref, bt_ref, o0, o1, o2, o3):
    z2 = _silu(jnp.dot(ea_ref[...], w1_ref[...], preferred_element_type=F32)
               + b1_ref[...])
    gbase = pl.program_id(0) * EB
    pad = (gbase + lax.broadcasted_iota(jnp.int32, (EB, 1), 0)) >= E
    outs = (o0, o1, o2, o3)
    for i in range(L):
        v = (jnp.dot(z2, wt_ref[i, :, :], preferred_element_type=F32)
             + bt_ref[i:i + 1, :])
        outs[i][...] = jnp.where(pad, -1e4, v)


@jax.jit
def _edge_cs(eap, edge_w1, edge_b1, wt, bt):
    es = pl.BlockSpec((EB, H), lambda i: (i, 0))
    return pl.pallas_call(
        _edge_cs_body,
        grid=(E_PAD // EB,),
        in_specs=[
            pl.BlockSpec((EB, DE), lambda i: (i, 0)),
            pl.BlockSpec((DE, H), lambda i: (0, 0)),
            pl.BlockSpec((1, H), lambda i: (0, 0)),
            pl.BlockSpec((L, H, H), lambda i: (0, 0, 0)),
            pl.BlockSpec((L, H), lambda i: (0, 0)),
        ],
        out_specs=(es, es, es, es),
        out_shape=tuple(jax.ShapeDtypeStruct((E_PAD, H), F32)
                        for _ in range(L)),
    )(eap, edge_w1, edge_b1.reshape(1, -1), wt, bt)


def _npre_body(h_ref, wa_ref, wb_ref, a_ref, b_ref):
    h = h_ref[...]
    a_ref[...] = jnp.dot(h, wa_ref[...], preferred_element_type=F32)
    b_ref[...] = jnp.dot(h, wb_ref[...], preferred_element_type=F32)


@jax.jit
def _node_pre(h, wa, wb):
    ns = pl.BlockSpec((NB, H), lambda i: (i, 0))
    ws = pl.BlockSpec((H, H), lambda i: (0, 0))
    return pl.pallas_call(
        _npre_body,
        grid=(N // NB,),
        in_specs=[ns, ws, ws],
        out_specs=(ns, ns),
        out_shape=(jax.ShapeDtypeStruct((N, H), F32),
                   jax.ShapeDtypeStruct((N, H), F32)),
    )(h, wa, wb)


def _ln_update(h, agg_ref, deg_ref, u1a, p_ref, q_ref, b1, u2, b2,
               g_ref, bt_ref):
    agg = agg_ref[0] + agg_ref[1]
    deg = deg_ref[0] + deg_ref[1]
    a = (jnp.dot(h, u1a[...], preferred_element_type=F32)
         + jnp.dot(agg, p_ref[...], preferred_element_type=F32)
         + deg * q_ref[...]
         + b1[...])
    u = jnp.dot(_silu(a), u2[...], preferred_element_type=F32) + b2[...]
    y = h + u
    mu = jnp.mean(y, axis=-1, keepdims=True)
    var = jnp.mean((y - mu) * (y - mu), axis=-1, keepdims=True)
    return (y - mu) / jnp.sqrt(var + 1e-5) * g_ref[...] + bt_ref[...]


def _upd_body(h_ref, agg_ref, deg_ref, u1a, p_ref, q_ref, b1, u2, b2,
              g_ref, bt_ref, o_ref):
    o_ref[...] = _ln_update(h_ref[...], agg_ref, deg_ref, u1a, p_ref, q_ref,
                            b1, u2, b2, g_ref, bt_ref)


_WS = pl.BlockSpec((H, H), lambda i: (0, 0))
_BS = pl.BlockSpec((1, H), lambda i: (0, 0))
_NS = pl.BlockSpec((NB, H), lambda i: (i, 0))
_AS = pl.BlockSpec((NC, NB, H), lambda i: (0, i, 0))


@jax.jit
def _update(h, agg2, deg2, u1, p, q, b1, u2, b2, g, bt):
    return pl.pallas_call(
        _upd_body,
        grid=(N // NB,),
        in_specs=[_NS, _AS, _AS, _WS, _WS, _BS, _BS, _WS, _BS, _BS, _BS],
        out_specs=_NS,
        out_shape=jax.ShapeDtypeStruct((N, H), F32),
    )(h, agg2, deg2, u1[:H], p, q.reshape(1, -1), b1.reshape(1, -1),
      u2, b2.reshape(1, -1), g.reshape(1, -1), bt.reshape(1, -1))


def _pool_body(h_ref, b_ref, sums_ref, cnts_ref, maxs_ref):
    i = pl.program_id(0)

    @pl.when(i == 0)
    def _init():
        sums_ref[...] = jnp.zeros((G, H), F32)
        cnts_ref[...] = jnp.zeros((G, H), F32)
        maxs_ref[...] = jnp.full((G, H), -1e30, F32)

    h = h_ref[...]
    b = b_ref[0, 0, :]
    gids = lax.broadcasted_iota(jnp.int32, (NB, G), 1)
    onehot = (b[:, None] == gids)
    oh_f = onehot.astype(F32)
    dn = (((0,), (0,)), ((), ()))
    sums_ref[...] += lax.dot_general(oh_f, h, dn, preferred_element_type=F32)
    cnts_ref[...] += lax.dot_general(oh_f, jnp.ones((NB, H), F32), dn,
                                     preferred_element_type=F32)
    parts = []
    for g in range(G):
        msk = b[:, None] == g
        parts.append(jnp.max(jnp.where(msk, h, -1e30), axis=0, keepdims=True))
    maxs_ref[...] = jnp.maximum(maxs_ref[...], jnp.concatenate(parts, axis=0))


@jax.jit
def _pool(h, batch3):
    gspec = pl.BlockSpec((G, H), lambda i: (0, 0))
    return pl.pallas_call(
        _pool_body,
        grid=(N // NB,),
        in_specs=[pl.BlockSpec((NB, H), lambda i: (i, 0)),
                  pl.BlockSpec((1, 1, NB), lambda i: (i, 0, 0))],
        out_specs=(gspec, gspec, gspec),
        out_shape=(jax.ShapeDtypeStruct((G, H), F32),
                   jax.ShapeDtypeStruct((G, H), F32),
                   jax.ShapeDtypeStruct((G, H), F32)),
    )(h, batch3)


def _head_body(sums_ref, cnts_ref, maxs_ref, w1a, w1b, b1, w2, b2,
               out_ref, gr_ref):
    cnts = cnts_ref[...]
    mean = sums_ref[...] / jnp.clip(cnts, 1.0, None)
    mx = jnp.where(cnts > 0, maxs_ref[...], 0.0)
    gr_ref[:, :H] = mean
    gr_ref[:, H:] = mx
    a = (jnp.dot(mean, w1a[...], preferred_element_type=F32)
         + jnp.dot(mx, w1b[...], preferred_element_type=F32)
         + b1[...])
    out_ref[...] = (jnp.dot(_silu(a), w2[...], preferred_element_type=F32)
                    + b2[...])


@jax.jit
def _head(sums, cnts, maxs, w1, b1, w2, b2):
    return pl.pallas_call(
        _head_body,
        out_shape=(jax.ShapeDtypeStruct((G, LAT), F32),
                   jax.ShapeDtypeStruct((G, 2 * H), F32)),
    )(sums, cnts, maxs, w1[:H], w1[H:], b1.reshape(1, -1), w2,
      b2.reshape(1, -1))


def kernel(x, edge_index, edge_attr, batch, node_w1, node_b1, node_w2,
           node_b2, edge_w1, edge_b1, edge_w2, edge_b2, msg_w1, msg_b1,
           msg_w2, msg_b2, upd_w1, upd_b1, upd_w2, upd_b2, ln_g, ln_b,
           out_w1, out_b1, out_w2, out_b2):
    srcp = jnp.pad(edge_index[0], (0, E_PAD - E)).reshape(-1, SPC * SCH)
    tgtu = jnp.pad(edge_index[1], (0, E_PAD - E)).reshape(-1, SCH)
    tgts64 = jnp.pad(edge_index[1], (0, E_PAD - E),
                     constant_values=N).reshape(-1, 64)
    eap = jnp.pad(edge_attr, ((0, E_PAD - E), (0, 0)))

    wt, btl, p_all, q_all = _weight_prep(msg_w1, msg_b1, msg_w2, msg_b2,
                                         upd_w1, edge_w2, edge_b2)
    h = _mlp2(x, node_w1, node_b1, node_w2, node_b2, NB)
    deg2 = _scatter_ones(tgts64)
    ces = _edge_cs(eap, edge_w1, edge_b1, wt, btl)

    for i in range(L):
        a_tab, b_tab = _node_pre(h, msg_w1[i][:H], msg_w1[i][H:2 * H])
        agg2 = _edge_sc(a_tab, b_tab, ces[i], srcp, tgtu)
        h = _update(h, agg2, deg2, upd_w1[i], p_all[i], q_all[i],
                    upd_b1[i], upd_w2[i], upd_b2[i], ln_g[i], ln_b[i])

    sums, cnts, maxs = _pool(h, batch.reshape(N // NB, 1, NB))
    return _head(sums, cnts, maxs, out_w1, out_b1, out_w2, out_b2)
